# baseline (speedup 1.0000x reference)
import jax
import jax.numpy as jnp
from jax.experimental import pallas as pl
from jax.experimental.pallas import tpu as pltpu

_B, _C = 4096, 1000
_CH = 512
_NCH = _B // _CH


def _chunk_part(x, yt, perm, ytsq, ytsql, lam):
    m = jnp.max(x, axis=1, keepdims=True)
    s = jnp.sum(jnp.exp(x - m), axis=1, keepdims=True)
    lse = m + jnp.log(s)

    biota = jax.lax.broadcasted_iota(jnp.int32, (_CH, 64), 1)
    u = jnp.where(biota == (perm & 63), 1.0, 0.0).astype(jnp.float32)
    v = jnp.where(biota == (perm >> 6), 1.0, 0.0).astype(jnp.float32)
    th = jax.lax.dot_general(u, ytsq, (((1,), (1,)), ((), ())),
                             preferred_element_type=jnp.float32)
    tl = jax.lax.dot_general(u, ytsql, (((1,), (1,)), ((), ())),
                             preferred_element_type=jnp.float32)
    l1h = jnp.sum(v * th, axis=1, keepdims=True)
    l1l = jnp.sum(v * tl, axis=1, keepdims=True)
    labels1 = (l1h * 256.0 + l1l).astype(jnp.int32)

    col = jax.lax.broadcasted_iota(jnp.int32, (_CH, _C), 1)
    p0 = jnp.sum(jnp.where(col == yt, x, 0.0), axis=1, keepdims=True)
    p1 = jnp.sum(jnp.where(col == labels1, x, 0.0), axis=1, keepdims=True)

    return (jnp.sum(lse, axis=0, keepdims=True)
            - lam * jnp.sum(p0, axis=0, keepdims=True)
            - (1.0 - lam) * jnp.sum(p1, axis=0, keepdims=True))


def _body(x_hbm, yt_ref, perm_ref, ytsq_ref, ytsql_ref, lam_ref, out_ref,
          xb, sems):
    def copy(k, slot):
        return pltpu.make_async_copy(
            x_hbm.at[pl.ds(k * _CH, _CH), :], xb.at[slot], sems.at[slot])

    copy(0, 0).start()
    lam = lam_ref[:, :]
    ytsq = ytsq_ref[:, :]
    ytsql = ytsql_ref[:, :]
    acc = jnp.zeros((1, 1), jnp.float32)
    for k in range(_NCH):
        slot = k % 2
        if k + 1 < _NCH:
            copy(k + 1, 1 - slot).start()
        copy(k, slot).wait()
        x = xb[slot]
        yt = yt_ref[pl.ds(k * _CH, _CH), :]
        perm = perm_ref[pl.ds(k * _CH, _CH), :]
        acc = acc + _chunk_part(x, yt, perm, ytsq, ytsql, lam)
    out_ref[:, :] = acc * (1.0 / _B)


def kernel(y_pred, y_true, perm_index, lam):
    lam_arr = jnp.asarray(lam, jnp.float32).reshape(1, 1)
    ytsq = (y_true >> 8).astype(jnp.float32).reshape(64, 64)
    ytsql = (y_true & 255).astype(jnp.float32).reshape(64, 64)
    out = pl.pallas_call(
        _body,
        in_specs=[
            pl.BlockSpec(memory_space=pl.ANY),
            pl.BlockSpec((_B, 1), lambda: (0, 0)),
            pl.BlockSpec((_B, 1), lambda: (0, 0)),
            pl.BlockSpec((64, 64), lambda: (0, 0)),
            pl.BlockSpec((64, 64), lambda: (0, 0)),
            pl.BlockSpec((1, 1), lambda: (0, 0)),
        ],
        out_specs=pl.BlockSpec((1, 1), lambda: (0, 0)),
        out_shape=jax.ShapeDtypeStruct((1, 1), jnp.float32),
        scratch_shapes=[
            pltpu.VMEM((2, _CH, _C), jnp.float32),
            pltpu.SemaphoreType.DMA((2,)),
        ],
    )(y_pred, y_true.reshape(_B, 1), perm_index.reshape(_B, 1), ytsq, ytsql,
      lam_arr)
    return out.reshape(())


# speedup vs baseline: 1.0708x; 1.0708x over previous
import jax
import jax.numpy as jnp
from jax.experimental import pallas as pl

_B, _C = 4096, 1000
_BR = 1024
_GRID = _B // _BR


def _body(x_ref, yt_ref, perm_ref, ytsq_ref, ytsql_ref, lam_ref, out_ref):
    i = pl.program_id(0)
    x = x_ref[:, :]
    m = jnp.max(x, axis=1, keepdims=True)
    s = jnp.sum(jnp.exp(x - m), axis=1, keepdims=True)
    lse = m + jnp.log(s)

    perm_blk = perm_ref[:, :]
    biota = jax.lax.broadcasted_iota(jnp.int32, (_BR, 64), 1)
    u = jnp.where(biota == (perm_blk & 63), 1.0, 0.0).astype(jnp.float32)
    v = jnp.where(biota == (perm_blk >> 6), 1.0, 0.0).astype(jnp.float32)
    yh = ytsq_ref[:, :]
    yl = ytsql_ref[:, :]
    th = jax.lax.dot_general(u, yh, (((1,), (1,)), ((), ())),
                             preferred_element_type=jnp.float32)
    tl = jax.lax.dot_general(u, yl, (((1,), (1,)), ((), ())),
                             preferred_element_type=jnp.float32)
    l1h = jnp.sum(v * th, axis=1, keepdims=True)
    l1l = jnp.sum(v * tl, axis=1, keepdims=True)
    labels1 = (l1h * 256.0 + l1l).astype(jnp.int32)

    col = jax.lax.broadcasted_iota(jnp.int32, (_BR, _C), 1)
    p0 = jnp.sum(jnp.where(col == yt_ref[:, :], x, 0.0), axis=1, keepdims=True)
    p1 = jnp.sum(jnp.where(col == labels1, x, 0.0), axis=1, keepdims=True)

    lam = lam_ref[:, :]
    part = (jnp.sum(lse, axis=0, keepdims=True)
            - lam * jnp.sum(p0, axis=0, keepdims=True)
            - (1.0 - lam) * jnp.sum(p1, axis=0, keepdims=True))

    @pl.when(i == 0)
    def _init():
        out_ref[:, :] = jnp.zeros_like(out_ref)

    out_ref[:, :] += part

    @pl.when(i == _GRID - 1)
    def _fin():
        out_ref[:, :] = out_ref[:, :] * (1.0 / _B)


def kernel(y_pred, y_true, perm_index, lam):
    lam_arr = jnp.asarray(lam, jnp.float32).reshape(1, 1)
    ytsq = (y_true >> 8).astype(jnp.float32).reshape(64, 64)
    ytsql = (y_true & 255).astype(jnp.float32).reshape(64, 64)
    out = pl.pallas_call(
        _body,
        grid=(_GRID,),
        in_specs=[
            pl.BlockSpec((_BR, _C), lambda i: (i, 0)),
            pl.BlockSpec((_BR, 1), lambda i: (i, 0)),
            pl.BlockSpec((_BR, 1), lambda i: (i, 0)),
            pl.BlockSpec((64, 64), lambda i: (0, 0)),
            pl.BlockSpec((64, 64), lambda i: (0, 0)),
            pl.BlockSpec((1, 1), lambda i: (0, 0)),
        ],
        out_specs=pl.BlockSpec((1, 1), lambda i: (0, 0)),
        out_shape=jax.ShapeDtypeStruct((1, 1), jnp.float32),
    )(y_pred, y_true.reshape(_B, 1), perm_index.reshape(_B, 1), ytsq, ytsql,
      lam_arr)
    return out.reshape(())
